# baseline (speedup 1.0000x reference)
import functools

import jax
import jax.numpy as jnp
from jax import lax
from jax.experimental import pallas as pl
from jax.experimental.pallas import tpu as pltpu
from jax.experimental.pallas import tpu_sc as plsc

N = 10000
E = 320000
D = 128
G = 64
NC = 2
NS = 16
NW = NC * NS
NPAD = 10240
STRIPE = NPAD // NS
EW = E // NW
CK = 80
NCH = EW // CK
RB = 5000
EPS = 1e-5

_mesh = plsc.VectorSubcoreMesh(
    core_axis_name="c", subcore_axis_name="s", num_cores=NC, num_subcores=NS)


NB = 5


@functools.partial(
    pl.kernel,
    out_type=jax.ShapeDtypeStruct((NC * NPAD,), jnp.float32),
    mesh=_mesh,
    scratch_types=[
        pltpu.VMEM((NCH, CK), jnp.int32),
        pltpu.VMEM((CK,), jnp.float32),
        pltpu.VMEM((STRIPE,), jnp.float32),
        pltpu.VMEM_SHARED((NPAD,), jnp.float32),
        [pltpu.SemaphoreType.DMA] * NB,
    ],
)
def _sc_degree(dst_hbm, out_hbm, didx_v, ones_v, zero_v, counts_sh, sems):
    c = lax.axis_index("c")
    s = lax.axis_index("s")
    w = c * NS + s

    def zfill(i, _):
        zero_v[pl.ds(i * 16, 16)] = jnp.zeros((16,), jnp.float32)
        return 0
    lax.fori_loop(0, STRIPE // 16, zfill, 0)

    def ofill(i, _):
        ones_v[pl.ds(i * 16, 16)] = jnp.ones((16,), jnp.float32)
        return 0
    lax.fori_loop(0, CK // 16, ofill, 0)

    pltpu.sync_copy(zero_v, counts_sh.at[pl.ds(s * STRIPE, STRIPE)])
    pltpu.sync_copy(dst_hbm.at[w], didx_v)
    plsc.subcore_barrier()

    def body(t, _):
        for b in range(NB):
            j = t * NB + b
            @pl.when(t > 0)
            def _():
                pltpu.make_async_copy(
                    ones_v, counts_sh.at[didx_v.at[j - NB]], sems[b]).wait()
            pltpu.async_copy(ones_v, counts_sh.at[didx_v.at[j]],
                             sems[b], add=True)
        return 0
    lax.fori_loop(0, NCH // NB, body, 0)
    for b in range(NB):
        pltpu.make_async_copy(
            ones_v, counts_sh.at[didx_v.at[NCH - NB + b]], sems[b]).wait()

    plsc.subcore_barrier()
    pltpu.sync_copy(counts_sh.at[pl.ds(s * STRIPE, STRIPE)],
                    out_hbm.at[pl.ds(c * NPAD + s * STRIPE, STRIPE)])


LG = 3
DH = D // NC
CKA = 128
EA = 337920
EPT = EA // NS
NCH2 = EPT // CKA


@functools.partial(
    pl.kernel,
    out_type=jax.ShapeDtypeStruct((NC, NPAD, DH), jnp.float32),
    mesh=_mesh,
    scratch_types=[
        pltpu.VMEM((NCH2, CKA), jnp.int32),
        pltpu.VMEM((NCH2, CKA), jnp.int32),
        [pltpu.VMEM((CKA, DH), jnp.float32)] * NB,
        pltpu.VMEM_SHARED((NPAD, DH), jnp.float32),
        [pltpu.SemaphoreType.DMA] * NB,
        [pltpu.SemaphoreType.DMA] * NB,
    ],
    compiler_params=pltpu.CompilerParams(use_tc_tiling_on_sc=False),
)
def _sc_agg(hs_hbm, src_hbm, dst_hbm, out_hbm,
            sidx_v, didx_v, rows, acc_sh, sem_g, sem_s):
    c = lax.axis_index("c")
    s = lax.axis_index("s")
    table = hs_hbm.at[c]

    def zrow(i, _):
        for k in range(DH // 16):
            rows[0][i, pl.ds(k * 16, 16)] = jnp.zeros((16,), jnp.float32)
        return 0
    lax.fori_loop(0, CKA, zrow, 0)
    for j in range(STRIPE // CKA):
        pltpu.sync_copy(rows[0], acc_sh.at[pl.ds(s * STRIPE + j * CKA, CKA)])

    pltpu.sync_copy(src_hbm.at[s], sidx_v)
    pltpu.sync_copy(dst_hbm.at[s], didx_v)
    plsc.subcore_barrier()

    for j in range(LG):
        pltpu.async_copy(table.at[sidx_v.at[j]], rows[j], sem_g[j])

    def body(t, _):
        for b in range(NB):
            j = t * NB + b
            bn = (b + LG) % NB
            pltpu.make_async_copy(
                table.at[sidx_v.at[j]], rows[b], sem_g[b]).wait()
            pltpu.async_copy(rows[b], acc_sh.at[didx_v.at[j]],
                             sem_s[b], add=True)

            @pl.when(j >= NB - LG)
            def _():
                pltpu.make_async_copy(
                    rows[bn], acc_sh.at[didx_v.at[j - (NB - LG)]],
                    sem_s[bn]).wait()

            @pl.when(j + LG < NCH2)
            def _():
                pltpu.async_copy(table.at[sidx_v.at[j + LG]],
                                 rows[bn], sem_g[bn])
        return 0
    lax.fori_loop(0, NCH2 // NB, body, 0)

    for j in range(NCH2 - (NB - LG), NCH2):
        pltpu.make_async_copy(
            rows[j % NB], acc_sh.at[didx_v.at[j]], sem_s[j % NB]).wait()

    plsc.subcore_barrier()
    pltpu.sync_copy(acc_sh.at[pl.ds(s * STRIPE, STRIPE)],
                    out_hbm.at[c, pl.ds(s * STRIPE, STRIPE)])


def _k_matmul(x, w):
    def body(x_ref, w_ref, h_ref):
        h_ref[...] = jnp.dot(x_ref[...], w_ref[...],
                             preferred_element_type=jnp.float32)
    return pl.pallas_call(
        body,
        grid=(N // RB,),
        in_specs=[
            pl.BlockSpec((RB, D), lambda i: (i, 0)),
            pl.BlockSpec((D, D), lambda i: (0, 0)),
        ],
        out_specs=pl.BlockSpec((RB, D), lambda i: (i, 0)),
        out_shape=jax.ShapeDtypeStruct((N, D), jnp.float32),
    )(x, w)


def _k_scale_split(h, p0, p1):
    def body(h_ref, p0_ref, p1_ref, hs_ref, dinv_ref):
        dinv = lax.rsqrt(p0_ref[...] + p1_ref[...] + 1.0)
        hs = h_ref[...] * dinv
        hs_ref[0] = hs[:, :DH]
        hs_ref[1] = hs[:, DH:]
        dinv_ref[...] = dinv
    return pl.pallas_call(
        body,
        grid=(N // RB,),
        in_specs=[
            pl.BlockSpec((RB, D), lambda i: (i, 0)),
            pl.BlockSpec((RB, 1), lambda i: (i, 0)),
            pl.BlockSpec((RB, 1), lambda i: (i, 0)),
        ],
        out_specs=[
            pl.BlockSpec((NC, RB, DH), lambda i: (0, i, 0)),
            pl.BlockSpec((RB, 1), lambda i: (i, 0)),
        ],
        out_shape=[
            jax.ShapeDtypeStruct((NC, N, DH), jnp.float32),
            jax.ShapeDtypeStruct((N, 1), jnp.float32),
        ],
    )(h, p0, p1)


def _bn_from_stats(sum_ref, ssq_ref):
    mean = sum_ref[...] * (1.0 / N)
    var = ssq_ref[...] * (1.0 / N) - mean * mean
    return mean, lax.rsqrt(var + EPS)


def _k_layer1_tail(q3, dinv, b, gamma, beta, w2):
    P = N // RB

    def body(q_ref, dinv_ref, b_ref, g_ref, be_ref, w_ref,
             out_ref, sum_v, ssq_v):
        pid = pl.program_id(0)
        pre = dinv_ref[...] * jnp.concatenate(
            [q_ref[0], q_ref[1]], axis=1) + b_ref[...]

        @pl.when(pid == 0)
        def _():
            sum_v[...] = jnp.zeros_like(sum_v)
            ssq_v[...] = jnp.zeros_like(ssq_v)

        @pl.when(pid < P)
        def _():
            sum_v[...] += jnp.sum(pre, axis=0, keepdims=True)
            ssq_v[...] += jnp.sum(pre * pre, axis=0, keepdims=True)

        @pl.when(pid >= P)
        def _():
            mean, inv = _bn_from_stats(sum_v, ssq_v)
            a = jnp.maximum((pre - mean) * inv * g_ref[...] + be_ref[...], 0.0)
            hs = jnp.dot(a, w_ref[...],
                         preferred_element_type=jnp.float32) * dinv_ref[...]
            out_ref[0] = hs[:, :DH]
            out_ref[1] = hs[:, DH:]

    return pl.pallas_call(
        body,
        grid=(2 * P,),
        in_specs=[
            pl.BlockSpec((NC, RB, DH), lambda i: (0, i % P, 0)),
            pl.BlockSpec((RB, 1), lambda i: (i % P, 0)),
            pl.BlockSpec((1, D), lambda i: (0, 0)),
            pl.BlockSpec((1, D), lambda i: (0, 0)),
            pl.BlockSpec((1, D), lambda i: (0, 0)),
            pl.BlockSpec((D, D), lambda i: (0, 0)),
        ],
        out_specs=pl.BlockSpec((NC, RB, DH),
                               lambda i: (0, jnp.maximum(i - P, 0), 0)),
        out_shape=jax.ShapeDtypeStruct((NC, N, DH), jnp.float32),
        scratch_shapes=[
            pltpu.VMEM((1, D), jnp.float32),
            pltpu.VMEM((1, D), jnp.float32),
        ],
    )(q3, dinv, b, gamma, beta, w2)


def _k_layer2_tail(q3, dinv, b, gamma, beta, batch3,
                   w_ih, b_ih, b_hh, w_fc, b_fc):
    P = N // RB

    def body(q_ref, dinv_ref, b_ref, g_ref, be_ref, bt_ref,
             wih_ref, bih_ref, bhh_ref, wfc_ref, bfc_ref,
             logp_ref, h1_ref, c1_ref, sum_v, ssq_v, pool_v, cnt_v):
        pid = pl.program_id(0)

        @pl.when(pid == 0)
        def _():
            sum_v[...] = jnp.zeros_like(sum_v)
            ssq_v[...] = jnp.zeros_like(ssq_v)
            pool_v[...] = jnp.zeros_like(pool_v)
            cnt_v[...] = jnp.zeros_like(cnt_v)

        @pl.when(pid < 2 * P)
        def _():
            pre = dinv_ref[...] * jnp.concatenate(
                [q_ref[0], q_ref[1]], axis=1) + b_ref[...]

            @pl.when(pid < P)
            def _():
                sum_v[...] += jnp.sum(pre, axis=0, keepdims=True)
                ssq_v[...] += jnp.sum(pre * pre, axis=0, keepdims=True)

            @pl.when(pid >= P)
            def _():
                mean, inv = _bn_from_stats(sum_v, ssq_v)
                a = jnp.maximum((pre - mean) * inv * g_ref[...] + be_ref[...],
                                0.0)
                gid = lax.broadcasted_iota(jnp.int32, (G, RB), 0)
                oh = (gid == bt_ref[0]).astype(jnp.float32)
                pool_v[...] += lax.dot_general(
                    oh, a, (((1,), (0,)), ((), ())),
                    preferred_element_type=jnp.float32)
                cnt_v[...] += jnp.sum(oh, axis=1, keepdims=True)

        @pl.when(pid == 2 * P)
        def _():
            pooled = pool_v[...] / jnp.maximum(cnt_v[...], 1.0)
            gates = lax.dot_general(pooled, wih_ref[...],
                                    (((1,), (1,)), ((), ())),
                                    preferred_element_type=jnp.float32)
            gates = gates + bih_ref[...] + bhh_ref[...]
            i_g = jax.nn.sigmoid(gates[:, 0 * D:1 * D])
            g_g = jnp.tanh(gates[:, 2 * D:3 * D])
            o_g = jax.nn.sigmoid(gates[:, 3 * D:4 * D])
            c1 = i_g * g_g
            h1 = o_g * jnp.tanh(c1)
            logits = lax.dot_general(h1, wfc_ref[...], (((1,), (1,)), ((), ())),
                                     preferred_element_type=jnp.float32)
            logits = logits + bfc_ref[...]
            m = jnp.max(logits, axis=1, keepdims=True)
            lse = jnp.log(jnp.sum(jnp.exp(logits - m), axis=1, keepdims=True))
            logp_ref[...] = logits - m - lse
            h1_ref[...] = h1
            c1_ref[...] = c1

    return pl.pallas_call(
        body,
        grid=(2 * P + 1,),
        in_specs=[
            pl.BlockSpec((NC, RB, DH), lambda i: (0, i % P, 0)),
            pl.BlockSpec((RB, 1), lambda i: (i % P, 0)),
            pl.BlockSpec((1, D), lambda i: (0, 0)),
            pl.BlockSpec((1, D), lambda i: (0, 0)),
            pl.BlockSpec((1, D), lambda i: (0, 0)),
            pl.BlockSpec((1, 1, RB), lambda i: (i % P, 0, 0)),
            pl.BlockSpec((4 * D, D), lambda i: (0, 0)),
            pl.BlockSpec((1, 4 * D), lambda i: (0, 0)),
            pl.BlockSpec((1, 4 * D), lambda i: (0, 0)),
            pl.BlockSpec((16, D), lambda i: (0, 0)),
            pl.BlockSpec((1, 16), lambda i: (0, 0)),
        ],
        out_specs=[
            pl.BlockSpec((G, 16), lambda i: (0, 0)),
            pl.BlockSpec((G, D), lambda i: (0, 0)),
            pl.BlockSpec((G, D), lambda i: (0, 0)),
        ],
        out_shape=[
            jax.ShapeDtypeStruct((G, 16), jnp.float32),
            jax.ShapeDtypeStruct((G, D), jnp.float32),
            jax.ShapeDtypeStruct((G, D), jnp.float32),
        ],
        scratch_shapes=[
            pltpu.VMEM((1, D), jnp.float32),
            pltpu.VMEM((1, D), jnp.float32),
            pltpu.VMEM((G, D), jnp.float32),
            pltpu.VMEM((G, 1), jnp.float32),
        ],
    )(q3, dinv, b, gamma, beta, batch3, w_ih, b_ih, b_hh, w_fc, b_fc)


def kernel(x, edge_index, batch, W1, b1, gamma1, beta1, W2, b2, gamma2,
           beta2, W_ih, W_hh, b_ih, b_hh, W_fc, b_fc):
    dst_d = edge_index[1].reshape(NW, NCH, CK)

    loops = jnp.arange(N, dtype=jnp.int32)
    padn = EA - E - N
    pad_src = jnp.arange(padn, dtype=jnp.int32) % N
    src_a = jnp.concatenate(
        [edge_index[0], loops, pad_src]
    ).reshape(NS, NCH2, CKA)
    pad_dst = N + (jnp.arange(padn, dtype=jnp.int32) % (NPAD - N))
    dst_a = jnp.concatenate(
        [edge_index[1], loops, pad_dst]
    ).reshape(NS, NCH2, CKA)

    h1 = _k_matmul(x, W1)
    degp = _sc_degree(dst_d)
    p0 = degp[0:N].reshape(N, 1)
    p1 = degp[NPAD:NPAD + N].reshape(N, 1)

    hs1, dinv = _k_scale_split(h1, p0, p1)

    agg1 = _sc_agg(hs1, src_a, dst_a)
    hs2 = _k_layer1_tail(agg1, dinv, b1.reshape(1, D),
                         gamma1.reshape(1, D), beta1.reshape(1, D), W2)

    agg2 = _sc_agg(hs2, src_a, dst_a)
    logp, h1, c1 = _k_layer2_tail(
        agg2, dinv, b2.reshape(1, D), gamma2.reshape(1, D),
        beta2.reshape(1, D), batch.reshape(N // RB, 1, RB),
        W_ih, b_ih.reshape(1, 4 * D), b_hh.reshape(1, 4 * D),
        W_fc, b_fc.reshape(1, 16))
    return (logp, (h1[None, :, :], c1[None, :, :]))


# speedup vs baseline: 1.0060x; 1.0060x over previous
import functools

import jax
import jax.numpy as jnp
from jax import lax
from jax.experimental import pallas as pl
from jax.experimental.pallas import tpu as pltpu
from jax.experimental.pallas import tpu_sc as plsc

N = 10000
E = 320000
D = 128
G = 64
NC = 2
NS = 16
NW = NC * NS
NPAD = 10240
STRIPE = NPAD // NS
EW = E // NW
CK = 80
NCH = EW // CK
RB = 5000
EPS = 1e-5

_mesh = plsc.VectorSubcoreMesh(
    core_axis_name="c", subcore_axis_name="s", num_cores=NC, num_subcores=NS)


NB = 5


@functools.partial(
    pl.kernel,
    out_type=jax.ShapeDtypeStruct((NC * NPAD,), jnp.float32),
    mesh=_mesh,
    scratch_types=[
        pltpu.VMEM((NCH, CK), jnp.int32),
        pltpu.VMEM((CK,), jnp.float32),
        pltpu.VMEM((STRIPE,), jnp.float32),
        pltpu.VMEM_SHARED((NPAD,), jnp.float32),
        [pltpu.SemaphoreType.DMA] * NB,
    ],
)
def _sc_degree(dst_hbm, out_hbm, didx_v, ones_v, zero_v, counts_sh, sems):
    c = lax.axis_index("c")
    s = lax.axis_index("s")
    w = c * NS + s

    def zfill(i, _):
        zero_v[pl.ds(i * 16, 16)] = jnp.zeros((16,), jnp.float32)
        return 0
    lax.fori_loop(0, STRIPE // 16, zfill, 0)

    def ofill(i, _):
        ones_v[pl.ds(i * 16, 16)] = jnp.ones((16,), jnp.float32)
        return 0
    lax.fori_loop(0, CK // 16, ofill, 0)

    pltpu.sync_copy(zero_v, counts_sh.at[pl.ds(s * STRIPE, STRIPE)])
    pltpu.sync_copy(dst_hbm.at[w], didx_v)
    plsc.subcore_barrier()

    def body(t, _):
        for b in range(NB):
            j = t * NB + b
            @pl.when(t > 0)
            def _():
                pltpu.make_async_copy(
                    ones_v, counts_sh.at[didx_v.at[j - NB]], sems[b]).wait()
            pltpu.async_copy(ones_v, counts_sh.at[didx_v.at[j]],
                             sems[b], add=True)
        return 0
    lax.fori_loop(0, NCH // NB, body, 0)
    for b in range(NB):
        pltpu.make_async_copy(
            ones_v, counts_sh.at[didx_v.at[NCH - NB + b]], sems[b]).wait()

    plsc.subcore_barrier()
    pltpu.sync_copy(counts_sh.at[pl.ds(s * STRIPE, STRIPE)],
                    out_hbm.at[pl.ds(c * NPAD + s * STRIPE, STRIPE)])


LG = 3
DH = D // NC
CKA = 128
EA = 337920
EPT = EA // NS
NCH2 = EPT // CKA


@functools.partial(
    pl.kernel,
    out_type=jax.ShapeDtypeStruct((NC, NPAD, DH), jnp.float32),
    mesh=_mesh,
    scratch_types=[
        pltpu.VMEM((NCH2, CKA), jnp.int32),
        pltpu.VMEM((NCH2, CKA), jnp.int32),
        [pltpu.VMEM((CKA, DH), jnp.float32)] * NB,
        pltpu.VMEM_SHARED((NPAD, DH), jnp.float32),
        [pltpu.SemaphoreType.DMA] * NB,
        [pltpu.SemaphoreType.DMA] * NB,
    ],
    compiler_params=pltpu.CompilerParams(use_tc_tiling_on_sc=False),
)
def _sc_agg(hs_hbm, src_hbm, dst_hbm, out_hbm,
            sidx_v, didx_v, rows, acc_sh, sem_g, sem_s):
    c = lax.axis_index("c")
    s = lax.axis_index("s")
    table = hs_hbm.at[c]

    def zrow(i, _):
        for k in range(DH // 16):
            rows[0][i, pl.ds(k * 16, 16)] = jnp.zeros((16,), jnp.float32)
        return 0
    lax.fori_loop(0, CKA, zrow, 0)
    for j in range(STRIPE // CKA):
        pltpu.sync_copy(rows[0], acc_sh.at[pl.ds(s * STRIPE + j * CKA, CKA)])

    pltpu.sync_copy(src_hbm.at[s], sidx_v)
    pltpu.sync_copy(dst_hbm.at[s], didx_v)
    plsc.subcore_barrier()

    for j in range(LG):
        pltpu.async_copy(table.at[sidx_v.at[j]], rows[j], sem_g[j])

    def body(t, _):
        for b in range(NB):
            j = t * NB + b
            bn = (b + LG) % NB
            pltpu.make_async_copy(
                table.at[sidx_v.at[j]], rows[b], sem_g[b]).wait()
            pltpu.async_copy(rows[b], acc_sh.at[didx_v.at[j]],
                             sem_s[b], add=True)

            @pl.when(j >= NB - LG)
            def _():
                pltpu.make_async_copy(
                    rows[bn], acc_sh.at[didx_v.at[j - (NB - LG)]],
                    sem_s[bn]).wait()

            @pl.when(j + LG < NCH2)
            def _():
                pltpu.async_copy(table.at[sidx_v.at[j + LG]],
                                 rows[bn], sem_g[bn])
        return 0
    lax.fori_loop(0, NCH2 // NB, body, 0)

    for j in range(NCH2 - (NB - LG), NCH2):
        pltpu.make_async_copy(
            rows[j % NB], acc_sh.at[didx_v.at[j]], sem_s[j % NB]).wait()

    plsc.subcore_barrier()
    pltpu.sync_copy(acc_sh.at[pl.ds(s * STRIPE, STRIPE)],
                    out_hbm.at[c, pl.ds(s * STRIPE, STRIPE)])


def _k_scale_matmul(x, w, p0, p1):
    def body(x_ref, w_ref, p0_ref, p1_ref, hs_ref, dinv_ref):
        dinv = lax.rsqrt(p0_ref[...] + p1_ref[...] + 1.0)
        h = jnp.dot(x_ref[...], w_ref[...], preferred_element_type=jnp.float32)
        hs = h * dinv
        hs_ref[0] = hs[:, :DH]
        hs_ref[1] = hs[:, DH:]
        dinv_ref[...] = dinv
    return pl.pallas_call(
        body,
        grid=(N // RB,),
        in_specs=[
            pl.BlockSpec((RB, D), lambda i: (i, 0)),
            pl.BlockSpec((D, D), lambda i: (0, 0)),
            pl.BlockSpec((RB, 1), lambda i: (i, 0)),
            pl.BlockSpec((RB, 1), lambda i: (i, 0)),
        ],
        out_specs=[
            pl.BlockSpec((NC, RB, DH), lambda i: (0, i, 0)),
            pl.BlockSpec((RB, 1), lambda i: (i, 0)),
        ],
        out_shape=[
            jax.ShapeDtypeStruct((NC, N, DH), jnp.float32),
            jax.ShapeDtypeStruct((N, 1), jnp.float32),
        ],
    )(x, w, p0, p1)


def _bn_from_stats(sum_ref, ssq_ref):
    mean = sum_ref[...] * (1.0 / N)
    var = ssq_ref[...] * (1.0 / N) - mean * mean
    return mean, lax.rsqrt(var + EPS)


def _k_layer1_tail(q3, dinv, b, gamma, beta, w2):
    P = N // RB

    def body(q_ref, dinv_ref, b_ref, g_ref, be_ref, w_ref,
             out_ref, sum_v, ssq_v):
        pid = pl.program_id(0)
        pre = dinv_ref[...] * jnp.concatenate(
            [q_ref[0], q_ref[1]], axis=1) + b_ref[...]

        @pl.when(pid == 0)
        def _():
            sum_v[...] = jnp.zeros_like(sum_v)
            ssq_v[...] = jnp.zeros_like(ssq_v)

        @pl.when(pid < P)
        def _():
            sum_v[...] += jnp.sum(pre, axis=0, keepdims=True)
            ssq_v[...] += jnp.sum(pre * pre, axis=0, keepdims=True)

        @pl.when(pid >= P)
        def _():
            mean, inv = _bn_from_stats(sum_v, ssq_v)
            a = jnp.maximum((pre - mean) * inv * g_ref[...] + be_ref[...], 0.0)
            hs = jnp.dot(a, w_ref[...],
                         preferred_element_type=jnp.float32) * dinv_ref[...]
            out_ref[0] = hs[:, :DH]
            out_ref[1] = hs[:, DH:]

    return pl.pallas_call(
        body,
        grid=(2 * P,),
        in_specs=[
            pl.BlockSpec((NC, RB, DH), lambda i: (0, i % P, 0)),
            pl.BlockSpec((RB, 1), lambda i: (i % P, 0)),
            pl.BlockSpec((1, D), lambda i: (0, 0)),
            pl.BlockSpec((1, D), lambda i: (0, 0)),
            pl.BlockSpec((1, D), lambda i: (0, 0)),
            pl.BlockSpec((D, D), lambda i: (0, 0)),
        ],
        out_specs=pl.BlockSpec((NC, RB, DH),
                               lambda i: (0, jnp.maximum(i - P, 0), 0)),
        out_shape=jax.ShapeDtypeStruct((NC, N, DH), jnp.float32),
        scratch_shapes=[
            pltpu.VMEM((1, D), jnp.float32),
            pltpu.VMEM((1, D), jnp.float32),
        ],
    )(q3, dinv, b, gamma, beta, w2)


def _k_layer2_tail(q3, dinv, b, gamma, beta, batch3,
                   w_ih, b_ih, b_hh, w_fc, b_fc):
    P = N // RB

    def body(q_ref, dinv_ref, b_ref, g_ref, be_ref, bt_ref,
             wih_ref, bih_ref, bhh_ref, wfc_ref, bfc_ref,
             logp_ref, h1_ref, c1_ref, sum_v, ssq_v, pool_v, cnt_v):
        pid = pl.program_id(0)

        @pl.when(pid == 0)
        def _():
            sum_v[...] = jnp.zeros_like(sum_v)
            ssq_v[...] = jnp.zeros_like(ssq_v)
            pool_v[...] = jnp.zeros_like(pool_v)
            cnt_v[...] = jnp.zeros_like(cnt_v)

        @pl.when(pid < 2 * P)
        def _():
            pre = dinv_ref[...] * jnp.concatenate(
                [q_ref[0], q_ref[1]], axis=1) + b_ref[...]

            @pl.when(pid < P)
            def _():
                sum_v[...] += jnp.sum(pre, axis=0, keepdims=True)
                ssq_v[...] += jnp.sum(pre * pre, axis=0, keepdims=True)

            @pl.when(pid >= P)
            def _():
                mean, inv = _bn_from_stats(sum_v, ssq_v)
                a = jnp.maximum((pre - mean) * inv * g_ref[...] + be_ref[...],
                                0.0)
                gid = lax.broadcasted_iota(jnp.int32, (G, RB), 0)
                oh = (gid == bt_ref[0]).astype(jnp.float32)
                pool_v[...] += lax.dot_general(
                    oh, a, (((1,), (0,)), ((), ())),
                    preferred_element_type=jnp.float32)
                cnt_v[...] += jnp.sum(oh, axis=1, keepdims=True)

        @pl.when(pid == 2 * P)
        def _():
            pooled = pool_v[...] / jnp.maximum(cnt_v[...], 1.0)
            gates = lax.dot_general(pooled, wih_ref[...],
                                    (((1,), (1,)), ((), ())),
                                    preferred_element_type=jnp.float32)
            gates = gates + bih_ref[...] + bhh_ref[...]
            i_g = jax.nn.sigmoid(gates[:, 0 * D:1 * D])
            g_g = jnp.tanh(gates[:, 2 * D:3 * D])
            o_g = jax.nn.sigmoid(gates[:, 3 * D:4 * D])
            c1 = i_g * g_g
            h1 = o_g * jnp.tanh(c1)
            logits = lax.dot_general(h1, wfc_ref[...], (((1,), (1,)), ((), ())),
                                     preferred_element_type=jnp.float32)
            logits = logits + bfc_ref[...]
            m = jnp.max(logits, axis=1, keepdims=True)
            lse = jnp.log(jnp.sum(jnp.exp(logits - m), axis=1, keepdims=True))
            logp_ref[...] = logits - m - lse
            h1_ref[...] = h1
            c1_ref[...] = c1

    return pl.pallas_call(
        body,
        grid=(2 * P + 1,),
        in_specs=[
            pl.BlockSpec((NC, RB, DH), lambda i: (0, i % P, 0)),
            pl.BlockSpec((RB, 1), lambda i: (i % P, 0)),
            pl.BlockSpec((1, D), lambda i: (0, 0)),
            pl.BlockSpec((1, D), lambda i: (0, 0)),
            pl.BlockSpec((1, D), lambda i: (0, 0)),
            pl.BlockSpec((1, 1, RB), lambda i: (i % P, 0, 0)),
            pl.BlockSpec((4 * D, D), lambda i: (0, 0)),
            pl.BlockSpec((1, 4 * D), lambda i: (0, 0)),
            pl.BlockSpec((1, 4 * D), lambda i: (0, 0)),
            pl.BlockSpec((16, D), lambda i: (0, 0)),
            pl.BlockSpec((1, 16), lambda i: (0, 0)),
        ],
        out_specs=[
            pl.BlockSpec((G, 16), lambda i: (0, 0)),
            pl.BlockSpec((G, D), lambda i: (0, 0)),
            pl.BlockSpec((G, D), lambda i: (0, 0)),
        ],
        out_shape=[
            jax.ShapeDtypeStruct((G, 16), jnp.float32),
            jax.ShapeDtypeStruct((G, D), jnp.float32),
            jax.ShapeDtypeStruct((G, D), jnp.float32),
        ],
        scratch_shapes=[
            pltpu.VMEM((1, D), jnp.float32),
            pltpu.VMEM((1, D), jnp.float32),
            pltpu.VMEM((G, D), jnp.float32),
            pltpu.VMEM((G, 1), jnp.float32),
        ],
    )(q3, dinv, b, gamma, beta, batch3, w_ih, b_ih, b_hh, w_fc, b_fc)


def kernel(x, edge_index, batch, W1, b1, gamma1, beta1, W2, b2, gamma2,
           beta2, W_ih, W_hh, b_ih, b_hh, W_fc, b_fc):
    dst_d = edge_index[1].reshape(NW, NCH, CK)

    loops = jnp.arange(N, dtype=jnp.int32)
    padn = EA - E - N
    pad_src = jnp.arange(padn, dtype=jnp.int32) % N
    src_a = jnp.concatenate(
        [edge_index[0], loops, pad_src]
    ).reshape(NS, NCH2, CKA)
    pad_dst = N + (jnp.arange(padn, dtype=jnp.int32) % (NPAD - N))
    dst_a = jnp.concatenate(
        [edge_index[1], loops, pad_dst]
    ).reshape(NS, NCH2, CKA)

    degp = _sc_degree(dst_d)
    p0 = degp[0:N].reshape(N, 1)
    p1 = degp[NPAD:NPAD + N].reshape(N, 1)

    hs1, dinv = _k_scale_matmul(x, W1, p0, p1)

    agg1 = _sc_agg(hs1, src_a, dst_a)
    hs2 = _k_layer1_tail(agg1, dinv, b1.reshape(1, D),
                         gamma1.reshape(1, D), beta1.reshape(1, D), W2)

    agg2 = _sc_agg(hs2, src_a, dst_a)
    logp, h1, c1 = _k_layer2_tail(
        agg2, dinv, b2.reshape(1, D), gamma2.reshape(1, D),
        beta2.reshape(1, D), batch.reshape(N // RB, 1, RB),
        W_ih, b_ih.reshape(1, 4 * D), b_hh.reshape(1, 4 * D),
        W_fc, b_fc.reshape(1, 16))
    return (logp, (h1[None, :, :], c1[None, :, :]))
